# Initial kernel scaffold; baseline (speedup 1.0000x reference)
import jax
import jax.numpy as jnp
from jax.experimental import pallas as pl


def kernel(x, evaluates, focuses):
    raise NotImplementedError("write your pallas kernel here")


# speedup vs baseline: 12.0476x; 12.0476x over previous
import functools

import jax
import jax.numpy as jnp
from jax import lax
from jax.experimental import pallas as pl
from jax.experimental.pallas import tpu as pltpu
from jax.experimental.pallas import tpu_sc as plsc

_NF = 8
_NB = 16
_L = 16
_NW = 32
_N = 204800
_P = _N // _NW
_C = 256
_NCHUNK = _P // _C


def _body(x_hbm, ev_hbm, fo_hbm, out_hbm, evv, fov, xin, outt):
    wid = lax.axis_index("s") * 2 + lax.axis_index("c")
    base0 = wid * _P
    pltpu.sync_copy(ev_hbm, evv)
    pltpu.sync_copy(fo_hbm, fov)
    iota = jnp.arange(_L, dtype=jnp.int32)

    def chunk_body(ci, carry):
        base = base0 + ci * _C
        pltpu.sync_copy(x_hbm.at[:, pl.ds(base, _C)], xin)

        def g_body(g, carry2):
            row = g * _L + iota
            for i in range(_NF):
                v = xin[i, pl.ds(g * _L, _L)]
                evec = evv[i, :]
                fvec = fov[i, :]
                dmin = jnp.full((_L,), 3.4e38, jnp.float32)
                dmax = jnp.full((_L,), -3.4e38, jnp.float32)
                fsel = jnp.zeros((_L,), jnp.float32)
                ds = []
                for j in range(_NB):
                    e = jnp.broadcast_to(evec[j], (_L,))
                    d = jnp.abs(v - e)
                    lt = d < dmin
                    fj = jnp.broadcast_to(fvec[j], (_L,))
                    fsel = jnp.where(lt, fj, fsel)
                    dmin = jnp.minimum(dmin, d)
                    dmax = jnp.maximum(dmax, d)
                    ds.append(d)
                nf = -fsel
                smax = jnp.maximum(dmin * nf, dmax * nf)
                acc = jnp.zeros((_L,), jnp.float32)
                ts = []
                for j in range(_NB):
                    t = jnp.exp(ds[j] * nf - smax)
                    acc = acc + t
                    ts.append(t)
                inv = 1.0 / acc
                for j in range(_NB):
                    col = jnp.full((_L,), i * _NB + j, jnp.int32)
                    plsc.store_scatter(outt, [row, col], ts[j] * inv)
            return carry2

        lax.fori_loop(0, _C // _L, g_body, 0)
        pltpu.sync_copy(outt, out_hbm.at[pl.ds(base, _C)])
        return carry

    lax.fori_loop(0, _NCHUNK, chunk_body, 0)


def kernel(x, evaluates, focuses):
    x2 = x.reshape(_N, _NF).T
    fo2 = focuses.reshape(_NF, _NB)
    mesh = plsc.VectorSubcoreMesh(core_axis_name="c", subcore_axis_name="s")
    k = functools.partial(
        pl.kernel,
        mesh=mesh,
        out_type=jax.ShapeDtypeStruct((_N, _NF * _NB), jnp.float32),
        scratch_types=[
            pltpu.VMEM((_NF, _NB), jnp.float32),
            pltpu.VMEM((_NF, _NB), jnp.float32),
            pltpu.VMEM((_NF, _C), jnp.float32),
            pltpu.VMEM((_C, _NF * _NB), jnp.float32),
        ],
        compiler_params=pltpu.CompilerParams(needs_layout_passes=False),
    )(_body)
    out = k(x2, evaluates, fo2)
    return out.reshape(x.shape[0], x.shape[1], _NF * _NB)
